# baseline (speedup 1.0000x reference)
import functools

import jax
import jax.numpy as jnp
from jax import lax
from jax.experimental import pallas as pl
from jax.experimental.pallas import tpu as pltpu
from jax.experimental.pallas import tpu_sc as plsc

NC = 2
NS = 16
NW = NC * NS
BB = 128
NBUF = 5
L = 16


@jax.jit
def _gather(xT, table):
    seq, nbatch = xT.shape
    dim = table.shape[1]
    nq = dim // L
    mesh = plsc.VectorSubcoreMesh(core_axis_name="c", subcore_axis_name="s")

    @functools.partial(
        pl.kernel,
        mesh=mesh,
        out_type=jax.ShapeDtypeStruct((seq, dim, nbatch), jnp.float32),
        compiler_params=pltpu.CompilerParams(
            use_tc_tiling_on_sc=False, needs_layout_passes=False
        ),
        scratch_types=[
            pltpu.VMEM((seq, BB), jnp.int32),
            pltpu.VMEM((NBUF, BB, dim), jnp.float32),
            pltpu.VMEM((2, dim, BB), jnp.float32),
        ]
        + [pltpu.SemaphoreType.DMA] * (NBUF + 2),
    )
    def body(xT_hbm, table_hbm, out_hbm, idx_v, rows_v, tbuf, *sems):
        gsems, wsems = sems[:NBUF], sems[NBUF:]
        wid = lax.axis_index("s") * NC + lax.axis_index("c")
        bcol = wid * BB
        pltpu.sync_copy(xT_hbm.at[:, pl.ds(bcol, BB)], idx_v)
        row_ids = [q * L + lax.iota(jnp.int32, L) for q in range(nq)]

        def fire(c, b):
            pltpu.async_copy(table_hbm.at[idx_v.at[c]], rows_v.at[b], gsems[b])

        def wait_gather(c, b):
            pltpu.make_async_copy(
                table_hbm.at[idx_v.at[c]], rows_v.at[b], gsems[b]
            ).wait()

        def wait_write(c, p):
            pltpu.make_async_copy(
                tbuf.at[p], out_hbm.at[c, :, pl.ds(bcol, BB)], wsems[p]
            ).wait()

        def process(c, b, p, first_pass):
            wait_gather(c, b)
            if not first_pass:
                wait_write(c - 2, p)

            def trans(j, carry):
                col = jnp.full((L,), 0, jnp.int32) + j
                for q in range(nq):
                    v = rows_v[b, j, pl.ds(q * L, L)]
                    plsc.store_scatter(tbuf.at[p], [row_ids[q], col], v)
                return carry

            lax.fori_loop(0, BB, trans, 0)
            pltpu.async_copy(
                tbuf.at[p], out_hbm.at[c, :, pl.ds(bcol, BB)], wsems[p]
            )

        for b in range(NBUF):
            fire(b, b)
        for b in range(2):
            process(b, b, b % 2, first_pass=True)
            fire(b + NBUF, b)
        for b in range(2, NBUF):
            process(b, b, b % 2, first_pass=False)
            fire(b + NBUF, b)

        def step(i, carry):
            for gg in range(2):
                for b in range(NBUF):
                    c = NBUF * (2 * i + gg + 1) + b
                    process(c, b, (NBUF * (gg + 1) + b) % 2, first_pass=False)
                    fire(c + NBUF, b)
            return carry

        nstep = (seq // NBUF - 2) // 2
        lax.fori_loop(0, nstep, step, 0)
        for b in range(NBUF):
            c = seq - NBUF + b
            process(c, b, c % 2, first_pass=False)
        for p in range(2):
            wait_write(seq - 2 + p, p)

    return body(xT, table)


def kernel(x, table):
    assert x.shape[0] == NW * BB
    xT = jnp.swapaxes(x, 0, 1).astype(jnp.int32)
    outT = _gather(xT, table)
    return jnp.transpose(outT, (2, 0, 1))


# speedup vs baseline: 1.0239x; 1.0239x over previous
import functools

import jax
import jax.numpy as jnp
from jax import lax
from jax.experimental import pallas as pl
from jax.experimental.pallas import tpu as pltpu
from jax.experimental.pallas import tpu_sc as plsc

NC = 2
NS = 16
NW = NC * NS
BB = 128
NBUF = 5
L = 16


@jax.jit
def _gather(xT, table):
    seq, nbatch = xT.shape
    dim = table.shape[1]
    nq = dim // L
    mesh = plsc.VectorSubcoreMesh(core_axis_name="c", subcore_axis_name="s")

    @functools.partial(
        pl.kernel,
        mesh=mesh,
        out_type=jax.ShapeDtypeStruct((seq, dim, nbatch), jnp.float32),
        compiler_params=pltpu.CompilerParams(
            use_tc_tiling_on_sc=False, needs_layout_passes=False
        ),
        scratch_types=[
            pltpu.VMEM((seq, BB), jnp.int32),
            pltpu.VMEM((NBUF, BB, dim), jnp.float32),
            pltpu.VMEM((2, dim, BB), jnp.float32),
        ]
        + [pltpu.SemaphoreType.DMA] * (NBUF + 2),
    )
    def body(xT_hbm, table_hbm, out_hbm, idx_v, rows_v, tbuf, *sems):
        gsems, wsems = sems[:NBUF], sems[NBUF:]
        wid = lax.axis_index("s") * NC + lax.axis_index("c")
        bcol = wid * BB
        pltpu.sync_copy(xT_hbm.at[:, pl.ds(bcol, BB)], idx_v)
        row_ids = [q * L + lax.iota(jnp.int32, L) for q in range(nq)]

        def fire(c, b):
            pltpu.async_copy(table_hbm.at[idx_v.at[c]], rows_v.at[b], gsems[b])

        def wait_gather(c, b):
            pltpu.make_async_copy(
                table_hbm.at[idx_v.at[c]], rows_v.at[b], gsems[b]
            ).wait()

        def wait_write(c, p):
            pltpu.make_async_copy(
                tbuf.at[p], out_hbm.at[c, :, pl.ds(bcol, BB)], wsems[p]
            ).wait()

        U = 8

        def process(c, b, p, first_pass):
            wait_gather(c, b)
            if not first_pass:
                wait_write(c - 2, p)

            def trans(i, cols):
                vs = []
                for u in range(U):
                    j = U * i + u
                    vs.append(
                        [rows_v[b, j, pl.ds(q * L, L)] for q in range(nq)]
                    )
                for u in range(U):
                    for q in range(nq):
                        plsc.store_scatter(
                            tbuf.at[p], [row_ids[q], cols[u]], vs[u][q]
                        )
                return tuple(cu + U for cu in cols)

            cols0 = tuple(jnp.full((L,), u, jnp.int32) for u in range(U))
            lax.fori_loop(0, BB // U, trans, cols0)
            pltpu.async_copy(
                tbuf.at[p], out_hbm.at[c, :, pl.ds(bcol, BB)], wsems[p]
            )

        for b in range(NBUF):
            fire(b, b)
        for b in range(2):
            process(b, b, b % 2, first_pass=True)
            fire(b + NBUF, b)
        for b in range(2, NBUF):
            process(b, b, b % 2, first_pass=False)
            fire(b + NBUF, b)

        def step(i, carry):
            for gg in range(2):
                for b in range(NBUF):
                    c = NBUF * (2 * i + gg + 1) + b
                    process(c, b, (NBUF * (gg + 1) + b) % 2, first_pass=False)
                    fire(c + NBUF, b)
            return carry

        nstep = (seq // NBUF - 2) // 2
        lax.fori_loop(0, nstep, step, 0)
        for b in range(NBUF):
            c = seq - NBUF + b
            process(c, b, c % 2, first_pass=False)
        for p in range(2):
            wait_write(seq - 2 + p, p)

    return body(xT, table)


def kernel(x, table):
    assert x.shape[0] == NW * BB
    xT = jnp.swapaxes(x, 0, 1).astype(jnp.int32)
    outT = _gather(xT, table)
    return jnp.transpose(outT, (2, 0, 1))


# speedup vs baseline: 1.7912x; 1.7493x over previous
import functools

import jax
import jax.numpy as jnp
from jax import lax
from jax.experimental import pallas as pl
from jax.experimental.pallas import tpu as pltpu
from jax.experimental.pallas import tpu_sc as plsc

NC = 2
NS = 16
NW = NC * NS
BB = 128
NBUF = 5
L = 16


@jax.jit
def _gather(xT, table):
    seq, nbatch = xT.shape
    dim = table.shape[1]
    nq = dim // L
    mesh = plsc.VectorSubcoreMesh(core_axis_name="c", subcore_axis_name="s")

    @functools.partial(
        pl.kernel,
        mesh=mesh,
        out_type=jax.ShapeDtypeStruct((seq, dim, nbatch), jnp.float32),
        compiler_params=pltpu.CompilerParams(
            use_tc_tiling_on_sc=False, needs_layout_passes=False
        ),
        scratch_types=[
            pltpu.VMEM((seq, BB), jnp.int32),
            pltpu.VMEM((NBUF, BB, dim), jnp.float32),
            pltpu.VMEM((2, dim, BB + 1), jnp.float32),
        ]
        + [pltpu.SemaphoreType.DMA] * (NBUF + 2),
    )
    def body(xT_hbm, table_hbm, out_hbm, idx_v, rows_v, tbuf, *sems):
        gsems, wsems = sems[:NBUF], sems[NBUF:]
        wid = lax.axis_index("s") * NC + lax.axis_index("c")
        bcol = wid * BB
        pltpu.sync_copy(xT_hbm.at[:, pl.ds(bcol, BB)], idx_v)
        row_ids = [q * L + lax.iota(jnp.int32, L) for q in range(nq)]

        def fire(c, b):
            pltpu.async_copy(table_hbm.at[idx_v.at[c]], rows_v.at[b], gsems[b])

        def wait_gather(c, b):
            pltpu.make_async_copy(
                table_hbm.at[idx_v.at[c]], rows_v.at[b], gsems[b]
            ).wait()

        def wait_write(c, p):
            pltpu.make_async_copy(
                tbuf.at[p, :, pl.ds(0, BB)], out_hbm.at[c, :, pl.ds(bcol, BB)], wsems[p]
            ).wait()

        U = 8

        def process(c, b, p, first_pass):
            wait_gather(c, b)
            if not first_pass:
                wait_write(c - 2, p)

            def trans(i, cols):
                vs = []
                for u in range(U):
                    j = U * i + u
                    vs.append(
                        [rows_v[b, j, pl.ds(q * L, L)] for q in range(nq)]
                    )
                for u in range(U):
                    for q in range(nq):
                        plsc.store_scatter(
                            tbuf.at[p], [row_ids[q], cols[u]], vs[u][q]
                        )
                return tuple(cu + U for cu in cols)

            cols0 = tuple(jnp.full((L,), u, jnp.int32) for u in range(U))
            lax.fori_loop(0, BB // U, trans, cols0)
            pltpu.async_copy(
                tbuf.at[p, :, pl.ds(0, BB)], out_hbm.at[c, :, pl.ds(bcol, BB)], wsems[p]
            )

        for b in range(NBUF):
            fire(b, b)
        for b in range(2):
            process(b, b, b % 2, first_pass=True)
            fire(b + NBUF, b)
        for b in range(2, NBUF):
            process(b, b, b % 2, first_pass=False)
            fire(b + NBUF, b)

        def step(i, carry):
            for gg in range(2):
                for b in range(NBUF):
                    c = NBUF * (2 * i + gg + 1) + b
                    process(c, b, (NBUF * (gg + 1) + b) % 2, first_pass=False)
                    fire(c + NBUF, b)
            return carry

        nstep = (seq // NBUF - 2) // 2
        lax.fori_loop(0, nstep, step, 0)
        for b in range(NBUF):
            c = seq - NBUF + b
            process(c, b, c % 2, first_pass=False)
        for p in range(2):
            wait_write(seq - 2 + p, p)

    return body(xT, table)


def kernel(x, table):
    assert x.shape[0] == NW * BB
    xT = jnp.swapaxes(x, 0, 1).astype(jnp.int32)
    outT = _gather(xT, table)
    return jnp.transpose(outT, (2, 0, 1))


# speedup vs baseline: 2.5024x; 1.3971x over previous
import functools

import jax
import jax.numpy as jnp
from jax import lax
from jax.experimental import pallas as pl
from jax.experimental.pallas import tpu as pltpu
from jax.experimental.pallas import tpu_sc as plsc

NC = 2
NS = 16
NW = NC * NS
BB = 128
NBUF = 5
L = 16


@jax.jit
def _gather(xT, table):
    seq, nbatch = xT.shape
    dim = table.shape[1]
    nq = dim // L
    mesh = plsc.VectorSubcoreMesh(core_axis_name="c", subcore_axis_name="s")

    @functools.partial(
        pl.kernel,
        mesh=mesh,
        out_type=jax.ShapeDtypeStruct(
            (seq, dim // 8, nbatch // BB, 8, BB), jnp.float32
        ),
        compiler_params=pltpu.CompilerParams(
            use_tc_tiling_on_sc=False, needs_layout_passes=False
        ),
        scratch_types=[
            pltpu.VMEM((seq, BB), jnp.int32),
            pltpu.VMEM((NBUF, BB, dim), jnp.float32),
            pltpu.VMEM((2, dim // 8, 8, BB + 1), jnp.float32),
        ]
        + [pltpu.SemaphoreType.DMA] * (NBUF + 2),
    )
    def body(xT_hbm, table_hbm, out_hbm, idx_v, rows_v, tbuf, *sems):
        gsems, wsems = sems[:NBUF], sems[NBUF:]
        wid = lax.axis_index("s") * NC + lax.axis_index("c")
        bcol = wid * BB
        pltpu.sync_copy(xT_hbm.at[:, pl.ds(bcol, BB)], idx_v)
        row_ids = [q * L + lax.iota(jnp.int32, L) for q in range(nq)]
        tile_ids = [r // 8 for r in row_ids]
        sub_ids = [r % 8 for r in row_ids]

        def fire(c, b):
            pltpu.async_copy(table_hbm.at[idx_v.at[c]], rows_v.at[b], gsems[b])

        def wait_gather(c, b):
            pltpu.make_async_copy(
                table_hbm.at[idx_v.at[c]], rows_v.at[b], gsems[b]
            ).wait()

        def wait_write(c, p):
            pltpu.make_async_copy(
                tbuf.at[p, :, :, pl.ds(0, BB)], out_hbm.at[c, :, wid], wsems[p]
            ).wait()

        U = 8

        def process(c, b, p, first_pass):
            wait_gather(c, b)
            if not first_pass:
                wait_write(c - 2, p)

            def trans(i, cols):
                vs = []
                for u in range(U):
                    j = U * i + u
                    vs.append(
                        [rows_v[b, j, pl.ds(q * L, L)] for q in range(nq)]
                    )
                for u in range(U):
                    for q in range(nq):
                        plsc.store_scatter(
                            tbuf.at[p], [tile_ids[q], sub_ids[q], cols[u]], vs[u][q]
                        )
                return tuple(cu + U for cu in cols)

            cols0 = tuple(jnp.full((L,), u, jnp.int32) for u in range(U))
            lax.fori_loop(0, BB // U, trans, cols0)
            pltpu.async_copy(
                tbuf.at[p, :, :, pl.ds(0, BB)], out_hbm.at[c, :, wid], wsems[p]
            )

        for b in range(NBUF):
            fire(b, b)
        for b in range(2):
            process(b, b, b % 2, first_pass=True)
            fire(b + NBUF, b)
        for b in range(2, NBUF):
            process(b, b, b % 2, first_pass=False)
            fire(b + NBUF, b)

        def step(i, carry):
            for gg in range(2):
                for b in range(NBUF):
                    c = NBUF * (2 * i + gg + 1) + b
                    process(c, b, (NBUF * (gg + 1) + b) % 2, first_pass=False)
                    fire(c + NBUF, b)
            return carry

        nstep = (seq // NBUF - 2) // 2
        lax.fori_loop(0, nstep, step, 0)
        for b in range(NBUF):
            c = seq - NBUF + b
            process(c, b, c % 2, first_pass=False)
        for p in range(2):
            wait_write(seq - 2 + p, p)

    return body(xT, table)


def kernel(x, table):
    assert x.shape[0] == NW * BB
    xT = jnp.swapaxes(x, 0, 1).astype(jnp.int32)
    out5 = _gather(xT, table)
    out = jnp.transpose(out5, (2, 4, 0, 1, 3))
    return out.reshape(x.shape[0], x.shape[1], table.shape[1])


# speedup vs baseline: 2.6136x; 1.0444x over previous
import functools

import jax
import jax.numpy as jnp
from jax import lax
from jax.experimental import pallas as pl
from jax.experimental.pallas import tpu as pltpu
from jax.experimental.pallas import tpu_sc as plsc

NC = 2
NS = 16
NW = NC * NS
BB = 128
NBUF = 5
L = 16


@functools.partial(jax.jit, static_argnums=(2,))
def _gather(xT, table, dim):
    seq, nbatch = xT.shape
    nq = dim // L
    mesh = plsc.VectorSubcoreMesh(core_axis_name="c", subcore_axis_name="s")

    @functools.partial(
        pl.kernel,
        mesh=mesh,
        out_type=jax.ShapeDtypeStruct(
            (seq, dim // 8, nbatch // BB, 8, BB), jnp.float32
        ),
        compiler_params=pltpu.CompilerParams(
            use_tc_tiling_on_sc=False, needs_layout_passes=False
        ),
        scratch_types=[
            pltpu.VMEM((seq, BB), jnp.int32),
            pltpu.VMEM((NBUF, BB, 128), jnp.float32),
            pltpu.VMEM((2, dim // 8, 8, BB + 1), jnp.float32),
        ]
        + [pltpu.SemaphoreType.DMA] * (NBUF + 2),
    )
    def body(xT_hbm, table_hbm, out_hbm, idx_v, rows_v, tbuf, *sems):
        gsems, wsems = sems[:NBUF], sems[NBUF:]
        wid = lax.axis_index("s") * NC + lax.axis_index("c")
        bcol = wid * BB
        pltpu.sync_copy(xT_hbm.at[:, pl.ds(bcol, BB)], idx_v)
        row_ids = [q * L + lax.iota(jnp.int32, L) for q in range(nq)]
        tile_ids = [r // 8 for r in row_ids]
        sub_ids = [r % 8 for r in row_ids]

        def fire(c, b):
            pltpu.async_copy(
                table_hbm.at[idx_v.at[c]], rows_v.at[b], gsems[b]
            )

        def wait_gather(c, b):
            pltpu.make_async_copy(
                table_hbm.at[idx_v.at[c]], rows_v.at[b], gsems[b]
            ).wait()

        def wait_write(c, p):
            pltpu.make_async_copy(
                tbuf.at[p, :, :, pl.ds(0, BB)], out_hbm.at[c, :, wid], wsems[p]
            ).wait()

        U = 8

        def process(c, b, p, first_pass):
            wait_gather(c, b)
            if not first_pass:
                wait_write(c - 2, p)

            def trans(i, cols):
                vs = []
                for u in range(U):
                    j = U * i + u
                    vs.append(
                        [rows_v[b, j, pl.ds(q * L, L)] for q in range(nq)]
                    )
                for u in range(U):
                    for q in range(nq):
                        plsc.store_scatter(
                            tbuf.at[p], [tile_ids[q], sub_ids[q], cols[u]], vs[u][q]
                        )
                return tuple(cu + U for cu in cols)

            cols0 = tuple(jnp.full((L,), u, jnp.int32) for u in range(U))
            lax.fori_loop(0, BB // U, trans, cols0)
            pltpu.async_copy(
                tbuf.at[p, :, :, pl.ds(0, BB)], out_hbm.at[c, :, wid], wsems[p]
            )

        for b in range(NBUF):
            fire(b, b)
        for b in range(2):
            process(b, b, b % 2, first_pass=True)
            fire(b + NBUF, b)
        for b in range(2, NBUF):
            process(b, b, b % 2, first_pass=False)
            fire(b + NBUF, b)

        def step(i, carry):
            for gg in range(2):
                for b in range(NBUF):
                    c = NBUF * (2 * i + gg + 1) + b
                    process(c, b, (NBUF * (gg + 1) + b) % 2, first_pass=False)
                    fire(c + NBUF, b)
            return carry

        nstep = (seq // NBUF - 2) // 2
        lax.fori_loop(0, nstep, step, 0)
        for b in range(NBUF):
            c = seq - NBUF + b
            process(c, b, c % 2, first_pass=False)
        for p in range(2):
            wait_write(seq - 2 + p, p)

    return body(xT, table)


def kernel(x, table):
    assert x.shape[0] == NW * BB
    xT = jnp.swapaxes(x, 0, 1).astype(jnp.int32)
    tp = jnp.pad(table, ((0, 0), (0, 128 - table.shape[1])))
    out5 = _gather(xT, tp, table.shape[1])
    out = jnp.transpose(out5, (2, 4, 0, 1, 3))
    return out.reshape(x.shape[0], x.shape[1], table.shape[1])
